# baseline (speedup 1.0000x reference)
import jax
import jax.numpy as jnp
from jax import lax
from jax.experimental import pallas as pl

LATENT = 128
N_TOKENS = 4096
BLK = 128
GRID = N_TOKENS // BLK


def _stage(y, w_ref, b_ref, id_ref, n_experts):
    ids = id_ref[0]
    y_bf = y.astype(jnp.bfloat16)
    zero = jnp.zeros_like(y_bf)

    def body(e, acc):
        ym = jnp.where(ids == e, y_bf, zero)
        return acc + jnp.dot(ym, w_ref[e], preferred_element_type=jnp.float32)

    acc = jnp.zeros((BLK, LATENT), jnp.float32)
    acc = lax.fori_loop(0, n_experts, body, acc, unroll=4)
    onehot = (ids == lax.broadcasted_iota(jnp.int32, (BLK, n_experts), 1))
    acc = acc + jnp.dot(onehot.astype(jnp.float32), b_ref[...],
                        preferred_element_type=jnp.float32)
    return jnp.maximum(acc, 0.0)


def _body(z_ref, wds_ref, bds_ref, was_ref, bas_ref, wdn_ref, bdn_ref,
          ids_ds_ref, ids_as_ref, ids_dn_ref, out_ref):
    y = z_ref[...]
    y = _stage(y, wds_ref, bds_ref, ids_ds_ref, 64)
    y = _stage(y, was_ref, bas_ref, ids_as_ref, 16)
    y = _stage(y, wdn_ref, bdn_ref, ids_dn_ref, 8)
    out_ref[...] = y


@jax.jit
def kernel(z, W_ds, b_ds, W_as, b_as, W_dn, b_dn, dataset_id, assay_id, donor_id):
    d = LATENT
    w_ds = W_ds.astype(jnp.bfloat16)
    w_as = W_as.astype(jnp.bfloat16)
    w_dn = W_dn.astype(jnp.bfloat16)
    ids_ds = dataset_id.astype(jnp.int32).reshape(GRID, BLK, 1)
    ids_as = assay_id.astype(jnp.int32).reshape(GRID, BLK, 1)
    ids_dn = donor_id.astype(jnp.int32).reshape(GRID, BLK, 1)

    full = lambda shape: pl.BlockSpec(shape, lambda i: (0,) * len(shape))
    out = pl.pallas_call(
        _body,
        grid=(GRID,),
        in_specs=[
            pl.BlockSpec((BLK, d), lambda i: (i, 0)),
            full(w_ds.shape), full(b_ds.shape),
            full(w_as.shape), full(b_as.shape),
            full(w_dn.shape), full(b_dn.shape),
            pl.BlockSpec((1, BLK, 1), lambda i: (i, 0, 0)),
            pl.BlockSpec((1, BLK, 1), lambda i: (i, 0, 0)),
            pl.BlockSpec((1, BLK, 1), lambda i: (i, 0, 0)),
        ],
        out_specs=pl.BlockSpec((BLK, d), lambda i: (i, 0)),
        out_shape=jax.ShapeDtypeStruct((N_TOKENS, d), jnp.float32),
    )(z, w_ds, b_ds, w_as, b_as, w_dn, b_dn, ids_ds, ids_as, ids_dn)
    return out


# speedup vs baseline: 2.0695x; 2.0695x over previous
import jax
import jax.numpy as jnp
from jax import lax
from jax.experimental import pallas as pl
from jax.experimental.pallas import tpu as pltpu
from jax.experimental.pallas import tpu_sc as plsc

N = 4096
D = 128
E1 = 64
T = 32
N1P = N + E1 * T
TILES = N1P // T
B1_BLK = 128
B1_GRID = N1P // B1_BLK
TC_BLK = 128
TC_GRID = N // TC_BLK
NSUB = 16
CHUNK = N // NSUB
NV = CHUNK // 16
NW2 = 32
CHUNK2 = N // NW2
NB = E1 // 16

_MESH = plsc.VectorSubcoreMesh(core_axis_name="c", subcore_axis_name="s")


def _splat(v, l):
    return v[jnp.full((16,), l, jnp.int32)]


def _pick64(vjs, g):
    res = jnp.zeros((16,), jnp.int32)
    for j in range(NB):
        gi = jnp.clip(g - 16 * j, 0, 15)
        m = (g >= 16 * j) & (g < 16 * (j + 1))
        res = res + jnp.where(m, vjs[j][gi], 0)
    return res


def _hs_cumsum(x):
    io16 = lax.iota(jnp.int32, 16)
    for s in (1, 2, 4, 8):
        sh = x[jnp.clip(io16 - s, 0, 15)]
        x = x + jnp.where(io16 >= s, sh, 0)
    return x


def _sc_sort_scatter(ds_id, z):
    def body(ds_hbm, z_hbm, zs_hbm, posback_hbm, eot_hbm,
             ids_ref, rank_ref, pos_ref, hist_ref, sums_ref,
             eot_ref, rows_ref, shist_ref, sem):
        c = lax.axis_index("c")
        wid = lax.axis_index("s")

        @pl.when(c == 0)
        def _():
            base = wid * CHUNK
            io16 = lax.iota(jnp.int32, 16)
            bv = [io16 + 16 * j for j in range(NB)]
            pltpu.sync_copy(ds_hbm.at[pl.ds(base, CHUNK)], ids_ref)

            H = [jnp.zeros((16,), jnp.int32) for _ in range(NB)]
            for k in range(NV):
                g = ids_ref[pl.ds(k * 16, 16)]
                prior = _pick64(H, g)

                def lane_body(l, carry):
                    acc = carry[0]
                    Hc = list(carry[1:])
                    gl = _splat(g, l)
                    acc = acc + jnp.where((io16 > l) & (g == gl), 1, 0)
                    for j in range(NB):
                        Hc[j] = Hc[j] + jnp.where(bv[j] == gl, 1, 0)
                    return (acc, *Hc)

                res = lax.fori_loop(
                    0, 16, lane_body,
                    (jnp.zeros((16,), jnp.int32), *H))
                H = list(res[1:])
                rank_ref[pl.ds(k * 16, 16)] = prior + res[0]

            for j in range(NB):
                hist_ref[pl.ds(j * 16, 16)] = H[j]
            pltpu.sync_copy(hist_ref, shist_ref.at[pl.ds(wid * E1, E1)])
            plsc.subcore_barrier()
            pltpu.sync_copy(shist_ref, sums_ref)

            tot = [jnp.zeros((16,), jnp.int32) for _ in range(NB)]
            off = [jnp.zeros((16,), jnp.int32) for _ in range(NB)]
            for w in range(NSUB):
                em = jnp.full((16,), jnp.where(w < wid, 1, 0))
                for j in range(NB):
                    h = sums_ref[pl.ds(w * E1 + j * 16, 16)]
                    tot[j] = tot[j] + h
                    off[j] = off[j] + h * em

            st, en = [], []
            run = jnp.zeros((16,), jnp.int32)
            for j in range(NB):
                pc = jnp.bitwise_and(tot[j] + (T - 1), -T)
                cs = _hs_cumsum(pc)
                s_j = cs - pc + run
                st.append(s_j)
                en.append(s_j + pc)
                run = run + _splat(cs, 15)

            @pl.when(wid < TILES // 16)
            def _():
                tstart = (io16 + wid * 16) * T
                acc = jnp.zeros((16,), jnp.int32)
                for j in range(NB):
                    for l in range(16):
                        es = _splat(en[j], l)
                        acc = acc + jnp.where(es <= tstart, 1, 0)
                eot_ref[...] = jnp.minimum(acc, E1 - 1)
                pltpu.sync_copy(eot_ref, eot_hbm.at[pl.ds(wid * 16, 16)])

            for k in range(NV):
                g = ids_ref[pl.ds(k * 16, 16)]
                pos_ref[pl.ds(k * 16, 16)] = (
                    _pick64(st, g) + _pick64(off, g)
                    + rank_ref[pl.ds(k * 16, 16)])

            pltpu.sync_copy(pos_ref, posback_hbm.at[pl.ds(base, CHUNK)])

            pltpu.sync_copy(z_hbm.at[pl.ds(base, CHUNK)], rows_ref)
            pltpu.async_copy(rows_ref, zs_hbm.at[pos_ref], sem).wait()

    f = pl.kernel(
        body,
        out_type=[
            jax.ShapeDtypeStruct((N1P, D), jnp.float32),
            jax.ShapeDtypeStruct((N,), jnp.int32),
            jax.ShapeDtypeStruct((TILES,), jnp.int32),
        ],
        mesh=_MESH,
        scratch_types=[
            pltpu.VMEM((CHUNK,), jnp.int32),
            pltpu.VMEM((CHUNK,), jnp.int32),
            pltpu.VMEM((CHUNK,), jnp.int32),
            pltpu.VMEM((E1,), jnp.int32),
            pltpu.VMEM((NSUB * E1,), jnp.int32),
            pltpu.VMEM((16,), jnp.int32),
            pltpu.VMEM((CHUNK, D), jnp.float32),
            pltpu.VMEM_SHARED((NSUB * E1,), jnp.int32),
            pltpu.SemaphoreType.DMA,
        ],
    )
    return f(ds_id, z)


def _sc_gather(posback, ys):
    def body(posback_hbm, ys_hbm, out_hbm, pos_ref, rows_ref, sem):
        wid = lax.axis_index("s") * 2 + lax.axis_index("c")
        base = wid * CHUNK2
        pltpu.sync_copy(posback_hbm.at[pl.ds(base, CHUNK2)], pos_ref)
        pltpu.async_copy(ys_hbm.at[pos_ref], rows_ref, sem).wait()
        pltpu.sync_copy(rows_ref, out_hbm.at[pl.ds(base, CHUNK2)])

    f = pl.kernel(
        body,
        out_type=jax.ShapeDtypeStruct((N, D), jnp.float32),
        mesh=_MESH,
        scratch_types=[
            pltpu.VMEM((CHUNK2,), jnp.int32),
            pltpu.VMEM((CHUNK2, D), jnp.float32),
            pltpu.SemaphoreType.DMA,
        ],
    )
    return f(posback, ys)


def _b1_body(eot_ref, zs_ref, w_ref, b_ref, out_ref):
    i = pl.program_id(0)
    for j in range(B1_BLK // T):
        e = eot_ref[(B1_BLK // T) * i + j]
        sub = zs_ref[pl.ds(T * j, T), :].astype(jnp.bfloat16)
        r = jnp.dot(sub, w_ref[e], preferred_element_type=jnp.float32)
        r = r + b_ref[pl.ds(e, 1), :]
        out_ref[pl.ds(T * j, T), :] = jnp.maximum(r, 0.0)


def _tc_grouped_stage1(eot, zs, w_bf, b):
    grid_spec = pltpu.PrefetchScalarGridSpec(
        num_scalar_prefetch=1,
        grid=(B1_GRID,),
        in_specs=[
            pl.BlockSpec((B1_BLK, D), lambda i, s: (i, 0)),
            pl.BlockSpec((E1, D, D), lambda i, s: (0, 0, 0)),
            pl.BlockSpec((E1, D), lambda i, s: (0, 0)),
        ],
        out_specs=pl.BlockSpec((B1_BLK, D), lambda i, s: (i, 0)),
    )
    return pl.pallas_call(
        _b1_body,
        grid_spec=grid_spec,
        out_shape=jax.ShapeDtypeStruct((N1P, D), jnp.float32),
    )(eot, zs, w_bf, b)


def _s23_stage(y, wt_ref, b_ref, id_ref, n_experts):
    ids = id_ref[0]
    h = jnp.dot(y.astype(jnp.bfloat16), wt_ref[...],
                preferred_element_type=jnp.float32)
    h3 = h.reshape(TC_BLK, n_experts, D)
    onehot = (ids == lax.broadcasted_iota(jnp.int32, (TC_BLK, n_experts), 1))
    sel = jnp.sum(h3 * onehot.astype(jnp.float32)[:, :, None], axis=1)
    out = sel + jnp.dot(onehot.astype(jnp.float32), b_ref[...],
                        preferred_element_type=jnp.float32)
    return jnp.maximum(out, 0.0)


def _s23_body(y_ref, was_ref, bas_ref, wdn_ref, bdn_ref,
              ids_as_ref, ids_dn_ref, out_ref):
    y = y_ref[...]
    y = _s23_stage(y, was_ref, bas_ref, ids_as_ref, 16)
    y = _s23_stage(y, wdn_ref, bdn_ref, ids_dn_ref, 8)
    out_ref[...] = y


def _tc_s23(y1, wt_as, b_as, wt_dn, b_dn, ids_as, ids_dn):
    full = lambda shape: pl.BlockSpec(shape, lambda i: (0,) * len(shape))
    return pl.pallas_call(
        _s23_body,
        grid=(TC_GRID,),
        in_specs=[
            pl.BlockSpec((TC_BLK, D), lambda i: (i, 0)),
            full(wt_as.shape), full(b_as.shape),
            full(wt_dn.shape), full(b_dn.shape),
            pl.BlockSpec((1, TC_BLK, 1), lambda i: (i, 0, 0)),
            pl.BlockSpec((1, TC_BLK, 1), lambda i: (i, 0, 0)),
        ],
        out_specs=pl.BlockSpec((TC_BLK, D), lambda i: (i, 0)),
        out_shape=jax.ShapeDtypeStruct((N, D), jnp.float32),
    )(y1, wt_as, b_as, wt_dn, b_dn, ids_as, ids_dn)


@jax.jit
def kernel(z, W_ds, b_ds, W_as, b_as, W_dn, b_dn, dataset_id, assay_id, donor_id):
    ds_id = dataset_id.astype(jnp.int32)
    zs, posback, eot = _sc_sort_scatter(ds_id, z)
    y1s = _tc_grouped_stage1(eot, zs, W_ds.astype(jnp.bfloat16), b_ds)
    y1 = _sc_gather(posback, y1s)
    wt_as = W_as.transpose(1, 0, 2).reshape(D, -1).astype(jnp.bfloat16)
    wt_dn = W_dn.transpose(1, 0, 2).reshape(D, -1).astype(jnp.bfloat16)
    ids_as = assay_id.astype(jnp.int32).reshape(TC_GRID, TC_BLK, 1)
    ids_dn = donor_id.astype(jnp.int32).reshape(TC_GRID, TC_BLK, 1)
    return _tc_s23(y1, wt_as, b_as, wt_dn, b_dn, ids_as, ids_dn)


# speedup vs baseline: 2.3619x; 1.1413x over previous
import jax
import jax.numpy as jnp
from jax import lax
from jax.experimental import pallas as pl
from jax.experimental.pallas import tpu as pltpu
from jax.experimental.pallas import tpu_sc as plsc

N = 4096
D = 128
E1 = 64
T = 32
N1P = N + E1 * T
TILES = N1P // T
B1_BLK = 128
B1_GRID = N1P // B1_BLK
TC_BLK = 128
TC_GRID = N // TC_BLK
NSUB = 16
CHUNK = N // NSUB
NV = CHUNK // 16
NW2 = 32
CHUNK2 = N // NW2
NB = E1 // 16

_MESH = plsc.VectorSubcoreMesh(core_axis_name="c", subcore_axis_name="s")


def _splat(v, l):
    return v[jnp.full((16,), l, jnp.int32)]


def _pick64(vjs, g):
    res = jnp.zeros((16,), jnp.int32)
    for j in range(NB):
        gi = jnp.clip(g - 16 * j, 0, 15)
        m = (g >= 16 * j) & (g < 16 * (j + 1))
        res = res + jnp.where(m, vjs[j][gi], 0)
    return res


def _hs_cumsum(x):
    io16 = lax.iota(jnp.int32, 16)
    for s in (1, 2, 4, 8):
        sh = x[jnp.clip(io16 - s, 0, 15)]
        x = x + jnp.where(io16 >= s, sh, 0)
    return x


def _sc_sort_scatter(ds_id, z):
    def body(ds_hbm, z_hbm, zs_hbm, posback_hbm, eot_hbm,
             ids_ref, rank_ref, pos_ref, hist_ref, sums_ref,
             eot_ref, rows_ref, shist_ref, sem):
        c = lax.axis_index("c")
        wid = lax.axis_index("s")

        @pl.when(c == 0)
        def _():
            base = wid * CHUNK
            io16 = lax.iota(jnp.int32, 16)
            bv = [io16 + 16 * j for j in range(NB)]
            pltpu.sync_copy(ds_hbm.at[pl.ds(base, CHUNK)], ids_ref)

            H = [jnp.zeros((16,), jnp.int32) for _ in range(NB)]
            for k in range(NV):
                g = ids_ref[pl.ds(k * 16, 16)]
                prior = _pick64(H, g)

                def lane_body(l, carry):
                    acc = carry[0]
                    Hc = list(carry[1:])
                    gl = _splat(g, l)
                    acc = acc + jnp.where((io16 > l) & (g == gl), 1, 0)
                    for j in range(NB):
                        Hc[j] = Hc[j] + jnp.where(bv[j] == gl, 1, 0)
                    return (acc, *Hc)

                res = lax.fori_loop(
                    0, 16, lane_body,
                    (jnp.zeros((16,), jnp.int32), *H))
                H = list(res[1:])
                rank_ref[pl.ds(k * 16, 16)] = prior + res[0]

            for j in range(NB):
                hist_ref[pl.ds(j * 16, 16)] = H[j]
            pltpu.sync_copy(hist_ref, shist_ref.at[pl.ds(wid * E1, E1)])
            plsc.subcore_barrier()
            pltpu.sync_copy(shist_ref, sums_ref)

            tot = [jnp.zeros((16,), jnp.int32) for _ in range(NB)]
            off = [jnp.zeros((16,), jnp.int32) for _ in range(NB)]
            for w in range(NSUB):
                em = jnp.full((16,), jnp.where(w < wid, 1, 0))
                for j in range(NB):
                    h = sums_ref[pl.ds(w * E1 + j * 16, 16)]
                    tot[j] = tot[j] + h
                    off[j] = off[j] + h * em

            st, en = [], []
            run = jnp.zeros((16,), jnp.int32)
            for j in range(NB):
                pc = jnp.bitwise_and(tot[j] + (T - 1), -T)
                cs = _hs_cumsum(pc)
                s_j = cs - pc + run
                st.append(s_j)
                en.append(s_j + pc)
                run = run + _splat(cs, 15)

            @pl.when(wid < TILES // 16)
            def _():
                tstart = (io16 + wid * 16) * T
                acc = jnp.zeros((16,), jnp.int32)
                for j in range(NB):
                    for l in range(16):
                        es = _splat(en[j], l)
                        acc = acc + jnp.where(es <= tstart, 1, 0)
                eot_ref[...] = jnp.minimum(acc, E1 - 1)
                pltpu.sync_copy(eot_ref, eot_hbm.at[pl.ds(wid * 16, 16)])

            for k in range(NV):
                g = ids_ref[pl.ds(k * 16, 16)]
                pos_ref[pl.ds(k * 16, 16)] = (
                    _pick64(st, g) + _pick64(off, g)
                    + rank_ref[pl.ds(k * 16, 16)])

            pltpu.sync_copy(pos_ref, posback_hbm.at[pl.ds(base, CHUNK)])

            pltpu.sync_copy(z_hbm.at[pl.ds(base, CHUNK)], rows_ref)
            pltpu.async_copy(rows_ref, zs_hbm.at[pos_ref], sem).wait()

    f = pl.kernel(
        body,
        out_type=[
            jax.ShapeDtypeStruct((N1P, D), jnp.float32),
            jax.ShapeDtypeStruct((N,), jnp.int32),
            jax.ShapeDtypeStruct((TILES,), jnp.int32),
        ],
        mesh=_MESH,
        scratch_types=[
            pltpu.VMEM((CHUNK,), jnp.int32),
            pltpu.VMEM((CHUNK,), jnp.int32),
            pltpu.VMEM((CHUNK,), jnp.int32),
            pltpu.VMEM((E1,), jnp.int32),
            pltpu.VMEM((NSUB * E1,), jnp.int32),
            pltpu.VMEM((16,), jnp.int32),
            pltpu.VMEM((CHUNK, D), jnp.float32),
            pltpu.VMEM_SHARED((NSUB * E1,), jnp.int32),
            pltpu.SemaphoreType.DMA,
        ],
    )
    return f(ds_id, z)


def _sc_gather(posback, ys):
    def body(posback_hbm, ys_hbm, out_hbm, pos_ref, rows_ref, sem):
        wid = lax.axis_index("s") * 2 + lax.axis_index("c")
        base = wid * CHUNK2
        pltpu.sync_copy(posback_hbm.at[pl.ds(base, CHUNK2)], pos_ref)
        pltpu.async_copy(ys_hbm.at[pos_ref], rows_ref, sem).wait()
        pltpu.sync_copy(rows_ref, out_hbm.at[pl.ds(base, CHUNK2)])

    f = pl.kernel(
        body,
        out_type=jax.ShapeDtypeStruct((N, D), jnp.float32),
        mesh=_MESH,
        scratch_types=[
            pltpu.VMEM((CHUNK2,), jnp.int32),
            pltpu.VMEM((CHUNK2, D), jnp.float32),
            pltpu.SemaphoreType.DMA,
        ],
    )
    return f(posback, ys)


def _b1_body(eot_ref, zs_ref, w_ref, b_ref, out_ref):
    i = pl.program_id(0)
    for j in range(B1_BLK // T):
        e = eot_ref[(B1_BLK // T) * i + j]
        sub = zs_ref[pl.ds(T * j, T), :].astype(jnp.bfloat16)
        r = jnp.dot(sub, w_ref[e], preferred_element_type=jnp.float32)
        r = r + b_ref[pl.ds(e, 1), :]
        out_ref[pl.ds(T * j, T), :] = jnp.maximum(r, 0.0)


def _tc_grouped_stage1(eot, zs, w_bf, b):
    grid_spec = pltpu.PrefetchScalarGridSpec(
        num_scalar_prefetch=1,
        grid=(B1_GRID,),
        in_specs=[
            pl.BlockSpec((B1_BLK, D), lambda i, s: (i, 0)),
            pl.BlockSpec((E1, D, D), lambda i, s: (0, 0, 0)),
            pl.BlockSpec((E1, D), lambda i, s: (0, 0)),
        ],
        out_specs=pl.BlockSpec((B1_BLK, D), lambda i, s: (i, 0)),
    )
    return pl.pallas_call(
        _b1_body,
        grid_spec=grid_spec,
        out_shape=jax.ShapeDtypeStruct((N1P, D), jnp.float32),
    )(eot, zs, w_bf, b)


def _s23_stage(y, ws_ref, b_ref, id_ref, n_experts):
    ids = id_ref[0]
    onehot = (ids == lax.broadcasted_iota(jnp.int32, (TC_BLK, n_experts), 1))
    colmap = lax.broadcasted_iota(
        jnp.int32, (TC_BLK, n_experts * D), 1) >> 7
    y_rep = jnp.concatenate([y.astype(jnp.bfloat16)] * n_experts, axis=1)
    yw = jnp.where(colmap == ids, y_rep, jnp.bfloat16(0))
    out = jnp.dot(yw, ws_ref[...], preferred_element_type=jnp.float32)
    out = out + jnp.dot(onehot.astype(jnp.float32), b_ref[...],
                        preferred_element_type=jnp.float32)
    return jnp.maximum(out, 0.0)


def _s23_body(y_ref, was_ref, bas_ref, wdn_ref, bdn_ref,
              ids_as_ref, ids_dn_ref, out_ref):
    y = y_ref[...]
    y = _s23_stage(y, was_ref, bas_ref, ids_as_ref, 16)
    y = _s23_stage(y, wdn_ref, bdn_ref, ids_dn_ref, 8)
    out_ref[...] = y


def _tc_s23(y1, wt_as, b_as, wt_dn, b_dn, ids_as, ids_dn):
    full = lambda shape: pl.BlockSpec(shape, lambda i: (0,) * len(shape))
    return pl.pallas_call(
        _s23_body,
        grid=(TC_GRID,),
        in_specs=[
            pl.BlockSpec((TC_BLK, D), lambda i: (i, 0)),
            full(wt_as.shape), full(b_as.shape),
            full(wt_dn.shape), full(b_dn.shape),
            pl.BlockSpec((1, TC_BLK, 1), lambda i: (i, 0, 0)),
            pl.BlockSpec((1, TC_BLK, 1), lambda i: (i, 0, 0)),
        ],
        out_specs=pl.BlockSpec((TC_BLK, D), lambda i: (i, 0)),
        out_shape=jax.ShapeDtypeStruct((N, D), jnp.float32),
    )(y1, wt_as, b_as, wt_dn, b_dn, ids_as, ids_dn)


@jax.jit
def kernel(z, W_ds, b_ds, W_as, b_as, W_dn, b_dn, dataset_id, assay_id, donor_id):
    ds_id = dataset_id.astype(jnp.int32)
    zs, posback, eot = _sc_sort_scatter(ds_id, z)
    y1s = _tc_grouped_stage1(eot, zs, W_ds.astype(jnp.bfloat16), b_ds)
    y1 = _sc_gather(posback, y1s)
    wt_as = W_as.reshape(-1, D).astype(jnp.bfloat16)
    wt_dn = W_dn.reshape(-1, D).astype(jnp.bfloat16)
    ids_as = assay_id.astype(jnp.int32).reshape(TC_GRID, TC_BLK, 1)
    ids_dn = donor_id.astype(jnp.int32).reshape(TC_GRID, TC_BLK, 1)
    return _tc_s23(y1, wt_as, b_as, wt_dn, b_dn, ids_as, ids_dn)


# speedup vs baseline: 3.0015x; 1.2708x over previous
import jax
import jax.numpy as jnp
from jax import lax
from jax.experimental import pallas as pl
from jax.experimental.pallas import tpu as pltpu
from jax.experimental.pallas import tpu_sc as plsc

N = 4096
D = 128
E1 = 64
T = 32
N1P = N + E1 * T
TILES = N1P // T
B1_BLK = 256
B1_GRID = N1P // B1_BLK
TC_BLK = 256
TC_GRID = N // TC_BLK
NSUB = 16
CHUNK = N // NSUB
NV = CHUNK // 16
NW2 = 32
CHUNK2 = N // NW2
NB = E1 // 16

_MESH = plsc.VectorSubcoreMesh(core_axis_name="c", subcore_axis_name="s")


def _splat(v, l):
    return v[jnp.full((16,), l, jnp.int32)]


def _pick64(vjs, g):
    res = jnp.zeros((16,), jnp.int32)
    for j in range(NB):
        gi = jnp.clip(g - 16 * j, 0, 15)
        m = (g >= 16 * j) & (g < 16 * (j + 1))
        res = res + jnp.where(m, vjs[j][gi], 0)
    return res


def _hs_cumsum(x):
    io16 = lax.iota(jnp.int32, 16)
    for s in (1, 2, 4, 8):
        sh = x[jnp.clip(io16 - s, 0, 15)]
        x = x + jnp.where(io16 >= s, sh, 0)
    return x


def _sc_sort_scatter(ds_id, z):
    def body(ds_hbm, z_hbm, zs_hbm, posback_hbm, eot_hbm,
             ids_ref, rank_ref, pos_ref, hist_ref, sums_ref,
             eot_ref, rows_ref, shist_ref, sem):
        c = lax.axis_index("c")
        wid = lax.axis_index("s")

        @pl.when(c == 0)
        def _():
            base = wid * CHUNK
            io16 = lax.iota(jnp.int32, 16)
            bv = [io16 + 16 * j for j in range(NB)]
            pltpu.sync_copy(ds_hbm.at[pl.ds(base, CHUNK)], ids_ref)

            H = [jnp.zeros((16,), jnp.int32) for _ in range(NB)]
            for k in range(NV):
                g = ids_ref[pl.ds(k * 16, 16)]
                prior = _pick64(H, g)

                def lane_body(l, carry):
                    acc = carry[0]
                    Hc = list(carry[1:])
                    gl = _splat(g, l)
                    acc = acc + jnp.where((io16 > l) & (g == gl), 1, 0)
                    for j in range(NB):
                        Hc[j] = Hc[j] + jnp.where(bv[j] == gl, 1, 0)
                    return (acc, *Hc)

                res = lax.fori_loop(
                    0, 16, lane_body,
                    (jnp.zeros((16,), jnp.int32), *H))
                H = list(res[1:])
                rank_ref[pl.ds(k * 16, 16)] = prior + res[0]

            for j in range(NB):
                hist_ref[pl.ds(j * 16, 16)] = H[j]
            pltpu.sync_copy(hist_ref, shist_ref.at[pl.ds(wid * E1, E1)])
            plsc.subcore_barrier()
            pltpu.sync_copy(shist_ref, sums_ref)

            tot = [jnp.zeros((16,), jnp.int32) for _ in range(NB)]
            off = [jnp.zeros((16,), jnp.int32) for _ in range(NB)]
            for w in range(NSUB):
                em = jnp.full((16,), jnp.where(w < wid, 1, 0))
                for j in range(NB):
                    h = sums_ref[pl.ds(w * E1 + j * 16, 16)]
                    tot[j] = tot[j] + h
                    off[j] = off[j] + h * em

            st, en = [], []
            run = jnp.zeros((16,), jnp.int32)
            for j in range(NB):
                pc = jnp.bitwise_and(tot[j] + (T - 1), -T)
                cs = _hs_cumsum(pc)
                s_j = cs - pc + run
                st.append(s_j)
                en.append(s_j + pc)
                run = run + _splat(cs, 15)

            @pl.when(wid < TILES // 16)
            def _():
                tstart = (io16 + wid * 16) * T
                acc = jnp.zeros((16,), jnp.int32)
                for j in range(NB):
                    for l in range(16):
                        es = _splat(en[j], l)
                        acc = acc + jnp.where(es <= tstart, 1, 0)
                eot_ref[...] = jnp.minimum(acc, E1 - 1)
                pltpu.sync_copy(eot_ref, eot_hbm.at[pl.ds(wid * 16, 16)])

            for k in range(NV):
                g = ids_ref[pl.ds(k * 16, 16)]
                pos_ref[pl.ds(k * 16, 16)] = (
                    _pick64(st, g) + _pick64(off, g)
                    + rank_ref[pl.ds(k * 16, 16)])

            pltpu.sync_copy(pos_ref, posback_hbm.at[pl.ds(base, CHUNK)])

            pltpu.sync_copy(z_hbm.at[pl.ds(base, CHUNK)], rows_ref)
            pltpu.async_copy(rows_ref, zs_hbm.at[pos_ref], sem).wait()

    f = pl.kernel(
        body,
        out_type=[
            jax.ShapeDtypeStruct((N1P, D), jnp.float32),
            jax.ShapeDtypeStruct((N,), jnp.int32),
            jax.ShapeDtypeStruct((TILES,), jnp.int32),
        ],
        mesh=_MESH,
        scratch_types=[
            pltpu.VMEM((CHUNK,), jnp.int32),
            pltpu.VMEM((CHUNK,), jnp.int32),
            pltpu.VMEM((CHUNK,), jnp.int32),
            pltpu.VMEM((E1,), jnp.int32),
            pltpu.VMEM((NSUB * E1,), jnp.int32),
            pltpu.VMEM((16,), jnp.int32),
            pltpu.VMEM((CHUNK, D), jnp.float32),
            pltpu.VMEM_SHARED((NSUB * E1,), jnp.int32),
            pltpu.SemaphoreType.DMA,
        ],
    )
    return f(ds_id, z)


def _sc_gather(posback, ys):
    def body(posback_hbm, ys_hbm, out_hbm, pos_ref, rows_ref, sem):
        wid = lax.axis_index("s") * 2 + lax.axis_index("c")
        base = wid * CHUNK2
        pltpu.sync_copy(posback_hbm.at[pl.ds(base, CHUNK2)], pos_ref)
        pltpu.async_copy(ys_hbm.at[pos_ref], rows_ref, sem).wait()
        pltpu.sync_copy(rows_ref, out_hbm.at[pl.ds(base, CHUNK2)])

    f = pl.kernel(
        body,
        out_type=jax.ShapeDtypeStruct((N, D), jnp.float32),
        mesh=_MESH,
        scratch_types=[
            pltpu.VMEM((CHUNK2,), jnp.int32),
            pltpu.VMEM((CHUNK2, D), jnp.float32),
            pltpu.SemaphoreType.DMA,
        ],
    )
    return f(posback, ys)


def _b1_body(eot_ref, zs_ref, w_ref, b_ref, out_ref):
    i = pl.program_id(0)
    for j in range(B1_BLK // T):
        e = eot_ref[(B1_BLK // T) * i + j]
        sub = zs_ref[pl.ds(T * j, T), :].astype(jnp.bfloat16)
        r = jnp.dot(sub, w_ref[e], preferred_element_type=jnp.float32)
        r = r + b_ref[pl.ds(e, 1), :]
        out_ref[pl.ds(T * j, T), :] = jnp.maximum(r, 0.0)


def _tc_grouped_stage1(eot, zs, w_bf, b):
    grid_spec = pltpu.PrefetchScalarGridSpec(
        num_scalar_prefetch=1,
        grid=(B1_GRID,),
        in_specs=[
            pl.BlockSpec((B1_BLK, D), lambda i, s: (i, 0)),
            pl.BlockSpec((E1, D, D), lambda i, s: (0, 0, 0)),
            pl.BlockSpec((E1, D), lambda i, s: (0, 0)),
        ],
        out_specs=pl.BlockSpec((B1_BLK, D), lambda i, s: (i, 0)),
    )
    return pl.pallas_call(
        _b1_body,
        grid_spec=grid_spec,
        out_shape=jax.ShapeDtypeStruct((N1P, D), jnp.float32),
    )(eot, zs, w_bf, b)


def _s23_stage(y, ws_ref, b_ref, id_ref, n_experts):
    ids = id_ref[0]
    onehot = (ids == lax.broadcasted_iota(jnp.int32, (TC_BLK, n_experts), 1))
    colmap = lax.broadcasted_iota(
        jnp.int32, (TC_BLK, n_experts * D), 1) >> 7
    y_rep = jnp.concatenate([y.astype(jnp.bfloat16)] * n_experts, axis=1)
    yw = jnp.where(colmap == ids, y_rep, jnp.bfloat16(0))
    out = jnp.dot(yw, ws_ref[...], preferred_element_type=jnp.float32)
    out = out + jnp.dot(onehot.astype(jnp.float32), b_ref[...],
                        preferred_element_type=jnp.float32)
    return jnp.maximum(out, 0.0)


def _s23_body(y_ref, was_ref, bas_ref, wdn_ref, bdn_ref,
              ids_as_ref, ids_dn_ref, out_ref):
    y = y_ref[...]
    y = _s23_stage(y, was_ref, bas_ref, ids_as_ref, 16)
    y = _s23_stage(y, wdn_ref, bdn_ref, ids_dn_ref, 8)
    out_ref[...] = y


def _tc_s23(y1, wt_as, b_as, wt_dn, b_dn, ids_as, ids_dn):
    full = lambda shape: pl.BlockSpec(shape, lambda i: (0,) * len(shape))
    return pl.pallas_call(
        _s23_body,
        grid=(TC_GRID,),
        in_specs=[
            pl.BlockSpec((TC_BLK, D), lambda i: (i, 0)),
            full(wt_as.shape), full(b_as.shape),
            full(wt_dn.shape), full(b_dn.shape),
            pl.BlockSpec((1, TC_BLK, 1), lambda i: (i, 0, 0)),
            pl.BlockSpec((1, TC_BLK, 1), lambda i: (i, 0, 0)),
        ],
        out_specs=pl.BlockSpec((TC_BLK, D), lambda i: (i, 0)),
        out_shape=jax.ShapeDtypeStruct((N, D), jnp.float32),
    )(y1, wt_as, b_as, wt_dn, b_dn, ids_as, ids_dn)


@jax.jit
def kernel(z, W_ds, b_ds, W_as, b_as, W_dn, b_dn, dataset_id, assay_id, donor_id):
    ds_id = dataset_id.astype(jnp.int32)
    zs, posback, eot = _sc_sort_scatter(ds_id, z)
    y1s = _tc_grouped_stage1(eot, zs, W_ds.astype(jnp.bfloat16), b_ds)
    y1 = _sc_gather(posback, y1s)
    wt_as = W_as.reshape(-1, D).astype(jnp.bfloat16)
    wt_dn = W_dn.reshape(-1, D).astype(jnp.bfloat16)
    ids_as = assay_id.astype(jnp.int32).reshape(TC_GRID, TC_BLK, 1)
    ids_dn = donor_id.astype(jnp.int32).reshape(TC_GRID, TC_BLK, 1)
    return _tc_s23(y1, wt_as, b_as, wt_dn, b_dn, ids_as, ids_dn)


# speedup vs baseline: 3.2854x; 1.0946x over previous
import jax
import jax.numpy as jnp
from jax import lax
from jax.experimental import pallas as pl
from jax.experimental.pallas import tpu as pltpu
from jax.experimental.pallas import tpu_sc as plsc

N = 4096
D = 128
E1 = 64
T = 32
N1P = N + E1 * T
TILES = N1P // T
B1_BLK = 512
B1_GRID = N1P // B1_BLK
TC_BLK = 512
TC_GRID = N // TC_BLK
NSUB = 16
CHUNK = N // NSUB
NV = CHUNK // 16
NW2 = 32
CHUNK2 = N // NW2
NB = E1 // 16

_MESH = plsc.VectorSubcoreMesh(core_axis_name="c", subcore_axis_name="s")


def _splat(v, l):
    return v[jnp.full((16,), l, jnp.int32)]


def _pick64(vjs, g):
    res = jnp.zeros((16,), jnp.int32)
    for j in range(NB):
        gi = jnp.clip(g - 16 * j, 0, 15)
        m = (g >= 16 * j) & (g < 16 * (j + 1))
        res = res + jnp.where(m, vjs[j][gi], 0)
    return res


def _hs_cumsum(x):
    io16 = lax.iota(jnp.int32, 16)
    for s in (1, 2, 4, 8):
        sh = x[jnp.clip(io16 - s, 0, 15)]
        x = x + jnp.where(io16 >= s, sh, 0)
    return x


def _sc_sort_scatter(ds_id, z):
    def body(ds_hbm, z_hbm, zs_hbm, posback_hbm, eot_hbm,
             ids_ref, rank_ref, pos_ref, hist_ref, sums_ref,
             eot_ref, rows_ref, shist_ref, sem):
        c = lax.axis_index("c")
        wid = lax.axis_index("s")

        @pl.when(c == 0)
        def _():
            base = wid * CHUNK
            io16 = lax.iota(jnp.int32, 16)
            bv = [io16 + 16 * j for j in range(NB)]
            pltpu.sync_copy(ds_hbm.at[pl.ds(base, CHUNK)], ids_ref)

            H = [jnp.zeros((16,), jnp.int32) for _ in range(NB)]
            for k in range(NV):
                g = ids_ref[pl.ds(k * 16, 16)]
                prior = _pick64(H, g)

                def lane_body(l, carry):
                    acc = carry[0]
                    Hc = list(carry[1:])
                    gl = _splat(g, l)
                    acc = acc + jnp.where((io16 > l) & (g == gl), 1, 0)
                    for j in range(NB):
                        Hc[j] = Hc[j] + jnp.where(bv[j] == gl, 1, 0)
                    return (acc, *Hc)

                res = lax.fori_loop(
                    0, 16, lane_body,
                    (jnp.zeros((16,), jnp.int32), *H))
                H = list(res[1:])
                rank_ref[pl.ds(k * 16, 16)] = prior + res[0]

            for j in range(NB):
                hist_ref[pl.ds(j * 16, 16)] = H[j]
            pltpu.sync_copy(hist_ref, shist_ref.at[pl.ds(wid * E1, E1)])
            plsc.subcore_barrier()
            pltpu.sync_copy(shist_ref, sums_ref)

            tot = [jnp.zeros((16,), jnp.int32) for _ in range(NB)]
            off = [jnp.zeros((16,), jnp.int32) for _ in range(NB)]
            for w in range(NSUB):
                em = jnp.full((16,), jnp.where(w < wid, 1, 0))
                for j in range(NB):
                    h = sums_ref[pl.ds(w * E1 + j * 16, 16)]
                    tot[j] = tot[j] + h
                    off[j] = off[j] + h * em

            st, en = [], []
            run = jnp.zeros((16,), jnp.int32)
            for j in range(NB):
                pc = jnp.bitwise_and(tot[j] + (T - 1), -T)
                cs = _hs_cumsum(pc)
                s_j = cs - pc + run
                st.append(s_j)
                en.append(s_j + pc)
                run = run + _splat(cs, 15)

            @pl.when(wid < TILES // 16)
            def _():
                tstart = (io16 + wid * 16) * T
                acc = jnp.zeros((16,), jnp.int32)
                for j in range(NB):
                    for l in range(16):
                        es = _splat(en[j], l)
                        acc = acc + jnp.where(es <= tstart, 1, 0)
                eot_ref[...] = jnp.minimum(acc, E1 - 1)
                pltpu.sync_copy(eot_ref, eot_hbm.at[pl.ds(wid * 16, 16)])

            for k in range(NV):
                g = ids_ref[pl.ds(k * 16, 16)]
                pos_ref[pl.ds(k * 16, 16)] = (
                    _pick64(st, g) + _pick64(off, g)
                    + rank_ref[pl.ds(k * 16, 16)])

            pltpu.sync_copy(pos_ref, posback_hbm.at[pl.ds(base, CHUNK)])

            pltpu.sync_copy(z_hbm.at[pl.ds(base, CHUNK)], rows_ref)
            pltpu.async_copy(rows_ref, zs_hbm.at[pos_ref], sem).wait()

    f = pl.kernel(
        body,
        out_type=[
            jax.ShapeDtypeStruct((N1P, D), jnp.float32),
            jax.ShapeDtypeStruct((N,), jnp.int32),
            jax.ShapeDtypeStruct((TILES,), jnp.int32),
        ],
        mesh=_MESH,
        scratch_types=[
            pltpu.VMEM((CHUNK,), jnp.int32),
            pltpu.VMEM((CHUNK,), jnp.int32),
            pltpu.VMEM((CHUNK,), jnp.int32),
            pltpu.VMEM((E1,), jnp.int32),
            pltpu.VMEM((NSUB * E1,), jnp.int32),
            pltpu.VMEM((16,), jnp.int32),
            pltpu.VMEM((CHUNK, D), jnp.float32),
            pltpu.VMEM_SHARED((NSUB * E1,), jnp.int32),
            pltpu.SemaphoreType.DMA,
        ],
    )
    return f(ds_id, z)


def _sc_gather(posback, ys):
    def body(posback_hbm, ys_hbm, out_hbm, pos_ref, rows_ref, sem):
        wid = lax.axis_index("s") * 2 + lax.axis_index("c")
        base = wid * CHUNK2
        pltpu.sync_copy(posback_hbm.at[pl.ds(base, CHUNK2)], pos_ref)
        pltpu.async_copy(ys_hbm.at[pos_ref], rows_ref, sem).wait()
        pltpu.sync_copy(rows_ref, out_hbm.at[pl.ds(base, CHUNK2)])

    f = pl.kernel(
        body,
        out_type=jax.ShapeDtypeStruct((N, D), jnp.float32),
        mesh=_MESH,
        scratch_types=[
            pltpu.VMEM((CHUNK2,), jnp.int32),
            pltpu.VMEM((CHUNK2, D), jnp.float32),
            pltpu.SemaphoreType.DMA,
        ],
    )
    return f(posback, ys)


def _b1_body(eot_ref, zs_ref, w_ref, b_ref, out_ref):
    i = pl.program_id(0)
    for j in range(B1_BLK // T):
        e = eot_ref[(B1_BLK // T) * i + j]
        sub = zs_ref[pl.ds(T * j, T), :].astype(jnp.bfloat16)
        r = jnp.dot(sub, w_ref[e], preferred_element_type=jnp.float32)
        r = r + b_ref[pl.ds(e, 1), :]
        out_ref[pl.ds(T * j, T), :] = jnp.maximum(r, 0.0)


def _tc_grouped_stage1(eot, zs, w_bf, b):
    grid_spec = pltpu.PrefetchScalarGridSpec(
        num_scalar_prefetch=1,
        grid=(B1_GRID,),
        in_specs=[
            pl.BlockSpec((B1_BLK, D), lambda i, s: (i, 0)),
            pl.BlockSpec((E1, D, D), lambda i, s: (0, 0, 0)),
            pl.BlockSpec((E1, D), lambda i, s: (0, 0)),
        ],
        out_specs=pl.BlockSpec((B1_BLK, D), lambda i, s: (i, 0)),
    )
    return pl.pallas_call(
        _b1_body,
        grid_spec=grid_spec,
        out_shape=jax.ShapeDtypeStruct((N1P, D), jnp.float32),
    )(eot, zs, w_bf, b)


def _s23_stage(y, ws_ref, b_ref, id_ref, n_experts):
    ids = id_ref[0]
    onehot = (ids == lax.broadcasted_iota(jnp.int32, (TC_BLK, n_experts), 1))
    colmap = lax.broadcasted_iota(
        jnp.int32, (TC_BLK, n_experts * D), 1) >> 7
    y_rep = jnp.concatenate([y.astype(jnp.bfloat16)] * n_experts, axis=1)
    yw = jnp.where(colmap == ids, y_rep, jnp.bfloat16(0))
    out = jnp.dot(yw, ws_ref[...], preferred_element_type=jnp.float32)
    out = out + jnp.dot(onehot.astype(jnp.float32), b_ref[...],
                        preferred_element_type=jnp.float32)
    return jnp.maximum(out, 0.0)


def _s23_body(y_ref, was_ref, bas_ref, wdn_ref, bdn_ref,
              ids_as_ref, ids_dn_ref, out_ref):
    y = y_ref[...]
    y = _s23_stage(y, was_ref, bas_ref, ids_as_ref, 16)
    y = _s23_stage(y, wdn_ref, bdn_ref, ids_dn_ref, 8)
    out_ref[...] = y


def _tc_s23(y1, wt_as, b_as, wt_dn, b_dn, ids_as, ids_dn):
    full = lambda shape: pl.BlockSpec(shape, lambda i: (0,) * len(shape))
    return pl.pallas_call(
        _s23_body,
        grid=(TC_GRID,),
        in_specs=[
            pl.BlockSpec((TC_BLK, D), lambda i: (i, 0)),
            full(wt_as.shape), full(b_as.shape),
            full(wt_dn.shape), full(b_dn.shape),
            pl.BlockSpec((1, TC_BLK, 1), lambda i: (i, 0, 0)),
            pl.BlockSpec((1, TC_BLK, 1), lambda i: (i, 0, 0)),
        ],
        out_specs=pl.BlockSpec((TC_BLK, D), lambda i: (i, 0)),
        out_shape=jax.ShapeDtypeStruct((N, D), jnp.float32),
    )(y1, wt_as, b_as, wt_dn, b_dn, ids_as, ids_dn)


@jax.jit
def kernel(z, W_ds, b_ds, W_as, b_as, W_dn, b_dn, dataset_id, assay_id, donor_id):
    ds_id = dataset_id.astype(jnp.int32)
    zs, posback, eot = _sc_sort_scatter(ds_id, z)
    y1s = _tc_grouped_stage1(eot, zs, W_ds.astype(jnp.bfloat16), b_ds)
    y1 = _sc_gather(posback, y1s)
    wt_as = W_as.reshape(-1, D).astype(jnp.bfloat16)
    wt_dn = W_dn.reshape(-1, D).astype(jnp.bfloat16)
    ids_as = assay_id.astype(jnp.int32).reshape(TC_GRID, TC_BLK, 1)
    ids_dn = donor_id.astype(jnp.int32).reshape(TC_GRID, TC_BLK, 1)
    return _tc_s23(y1, wt_as, b_as, wt_dn, b_dn, ids_as, ids_dn)


# speedup vs baseline: 3.4665x; 1.0551x over previous
import jax
import jax.numpy as jnp
from jax import lax
from jax.experimental import pallas as pl
from jax.experimental.pallas import tpu as pltpu
from jax.experimental.pallas import tpu_sc as plsc

N = 4096
D = 128
E1 = 64
T = 32
N1P = N + E1 * T
TILES = N1P // T
B1_BLK = 1024
B1_GRID = N1P // B1_BLK
TC_BLK = 512
TC_GRID = N // TC_BLK
NSUB = 16
CHUNK = N // NSUB
NV = CHUNK // 16
NW2 = 32
CHUNK2 = N // NW2
NB = E1 // 16

_MESH = plsc.VectorSubcoreMesh(core_axis_name="c", subcore_axis_name="s")


def _splat(v, l):
    return v[jnp.full((16,), l, jnp.int32)]


def _pick64(vjs, g):
    res = jnp.zeros((16,), jnp.int32)
    for j in range(NB):
        gi = jnp.clip(g - 16 * j, 0, 15)
        m = (g >= 16 * j) & (g < 16 * (j + 1))
        res = res + jnp.where(m, vjs[j][gi], 0)
    return res


def _hs_cumsum(x):
    io16 = lax.iota(jnp.int32, 16)
    for s in (1, 2, 4, 8):
        sh = x[jnp.clip(io16 - s, 0, 15)]
        x = x + jnp.where(io16 >= s, sh, 0)
    return x


def _sc_sort_scatter(ds_id, z):
    def body(ds_hbm, z_hbm, zs_hbm, posback_hbm, eot_hbm,
             ids_ref, rank_ref, pos_ref, hist_ref, sums_ref,
             eot_ref, rows_ref, shist_ref, sem):
        c = lax.axis_index("c")
        wid = lax.axis_index("s")

        @pl.when(c == 0)
        def _():
            base = wid * CHUNK
            io16 = lax.iota(jnp.int32, 16)
            bv = [io16 + 16 * j for j in range(NB)]
            pltpu.sync_copy(ds_hbm.at[pl.ds(base, CHUNK)], ids_ref)

            H = [jnp.zeros((16,), jnp.int32) for _ in range(NB)]
            for k in range(NV):
                g = ids_ref[pl.ds(k * 16, 16)]
                prior = _pick64(H, g)

                def lane_body(l, carry):
                    acc = carry[0]
                    Hc = list(carry[1:])
                    gl = _splat(g, l)
                    acc = acc + jnp.where((io16 > l) & (g == gl), 1, 0)
                    for j in range(NB):
                        Hc[j] = Hc[j] + jnp.where(bv[j] == gl, 1, 0)
                    return (acc, *Hc)

                res = lax.fori_loop(
                    0, 16, lane_body,
                    (jnp.zeros((16,), jnp.int32), *H))
                H = list(res[1:])
                rank_ref[pl.ds(k * 16, 16)] = prior + res[0]

            for j in range(NB):
                hist_ref[pl.ds(j * 16, 16)] = H[j]
            pltpu.sync_copy(hist_ref, shist_ref.at[pl.ds(wid * E1, E1)])
            plsc.subcore_barrier()
            pltpu.sync_copy(shist_ref, sums_ref)

            tot = [jnp.zeros((16,), jnp.int32) for _ in range(NB)]
            off = [jnp.zeros((16,), jnp.int32) for _ in range(NB)]
            for w in range(NSUB):
                em = jnp.full((16,), jnp.where(w < wid, 1, 0))
                for j in range(NB):
                    h = sums_ref[pl.ds(w * E1 + j * 16, 16)]
                    tot[j] = tot[j] + h
                    off[j] = off[j] + h * em

            st, en = [], []
            run = jnp.zeros((16,), jnp.int32)
            for j in range(NB):
                pc = jnp.bitwise_and(tot[j] + (T - 1), -T)
                cs = _hs_cumsum(pc)
                s_j = cs - pc + run
                st.append(s_j)
                en.append(s_j + pc)
                run = run + _splat(cs, 15)

            @pl.when(wid < TILES // 16)
            def _():
                tstart = (io16 + wid * 16) * T
                acc = jnp.zeros((16,), jnp.int32)
                for j in range(NB):
                    for l in range(16):
                        es = _splat(en[j], l)
                        acc = acc + jnp.where(es <= tstart, 1, 0)
                eot_ref[...] = jnp.minimum(acc, E1 - 1)
                pltpu.sync_copy(eot_ref, eot_hbm.at[pl.ds(wid * 16, 16)])

            for k in range(NV):
                g = ids_ref[pl.ds(k * 16, 16)]
                pos_ref[pl.ds(k * 16, 16)] = (
                    _pick64(st, g) + _pick64(off, g)
                    + rank_ref[pl.ds(k * 16, 16)])

            pltpu.sync_copy(pos_ref, posback_hbm.at[pl.ds(base, CHUNK)])

            pltpu.sync_copy(z_hbm.at[pl.ds(base, CHUNK)], rows_ref)
            pltpu.async_copy(rows_ref, zs_hbm.at[pos_ref], sem).wait()

    f = pl.kernel(
        body,
        out_type=[
            jax.ShapeDtypeStruct((N1P, D), jnp.float32),
            jax.ShapeDtypeStruct((N,), jnp.int32),
            jax.ShapeDtypeStruct((TILES,), jnp.int32),
        ],
        mesh=_MESH,
        scratch_types=[
            pltpu.VMEM((CHUNK,), jnp.int32),
            pltpu.VMEM((CHUNK,), jnp.int32),
            pltpu.VMEM((CHUNK,), jnp.int32),
            pltpu.VMEM((E1,), jnp.int32),
            pltpu.VMEM((NSUB * E1,), jnp.int32),
            pltpu.VMEM((16,), jnp.int32),
            pltpu.VMEM((CHUNK, D), jnp.float32),
            pltpu.VMEM_SHARED((NSUB * E1,), jnp.int32),
            pltpu.SemaphoreType.DMA,
        ],
    )
    return f(ds_id, z)


def _sc_gather(posback, ys):
    def body(posback_hbm, ys_hbm, out_hbm, pos_ref, rows_ref, sem):
        wid = lax.axis_index("s") * 2 + lax.axis_index("c")
        base = wid * CHUNK2
        pltpu.sync_copy(posback_hbm.at[pl.ds(base, CHUNK2)], pos_ref)
        pltpu.async_copy(ys_hbm.at[pos_ref], rows_ref, sem).wait()
        pltpu.sync_copy(rows_ref, out_hbm.at[pl.ds(base, CHUNK2)])

    f = pl.kernel(
        body,
        out_type=jax.ShapeDtypeStruct((N, D), jnp.float32),
        mesh=_MESH,
        scratch_types=[
            pltpu.VMEM((CHUNK2,), jnp.int32),
            pltpu.VMEM((CHUNK2, D), jnp.float32),
            pltpu.SemaphoreType.DMA,
        ],
    )
    return f(posback, ys)


def _b1_body(eot_ref, zs_ref, w_ref, b_ref, out_ref):
    i = pl.program_id(0)
    for j in range(B1_BLK // T):
        e = eot_ref[(B1_BLK // T) * i + j]
        sub = zs_ref[pl.ds(T * j, T), :].astype(jnp.bfloat16)
        r = jnp.dot(sub, w_ref[e], preferred_element_type=jnp.float32)
        r = r + b_ref[pl.ds(e, 1), :]
        out_ref[pl.ds(T * j, T), :] = jnp.maximum(r, 0.0)


def _tc_grouped_stage1(eot, zs, w_bf, b):
    grid_spec = pltpu.PrefetchScalarGridSpec(
        num_scalar_prefetch=1,
        grid=(B1_GRID,),
        in_specs=[
            pl.BlockSpec((B1_BLK, D), lambda i, s: (i, 0)),
            pl.BlockSpec((E1, D, D), lambda i, s: (0, 0, 0)),
            pl.BlockSpec((E1, D), lambda i, s: (0, 0)),
        ],
        out_specs=pl.BlockSpec((B1_BLK, D), lambda i, s: (i, 0)),
    )
    return pl.pallas_call(
        _b1_body,
        grid_spec=grid_spec,
        out_shape=jax.ShapeDtypeStruct((N1P, D), jnp.float32),
    )(eot, zs, w_bf, b)


def _s23_stage(y, ws_ref, b_ref, id_ref, n_experts):
    ids = id_ref[0]
    onehot = (ids == lax.broadcasted_iota(jnp.int32, (TC_BLK, n_experts), 1))
    colmap = lax.broadcasted_iota(
        jnp.int32, (TC_BLK, n_experts * D), 1) >> 7
    y_rep = jnp.concatenate([y.astype(jnp.bfloat16)] * n_experts, axis=1)
    yw = jnp.where(colmap == ids, y_rep, jnp.bfloat16(0))
    out = jnp.dot(yw, ws_ref[...], preferred_element_type=jnp.float32)
    out = out + jnp.dot(onehot.astype(jnp.float32), b_ref[...],
                        preferred_element_type=jnp.float32)
    return jnp.maximum(out, 0.0)


def _s23_body(y_ref, was_ref, bas_ref, wdn_ref, bdn_ref,
              ids_as_ref, ids_dn_ref, out_ref):
    y = y_ref[...]
    y = _s23_stage(y, was_ref, bas_ref, ids_as_ref, 16)
    y = _s23_stage(y, wdn_ref, bdn_ref, ids_dn_ref, 8)
    out_ref[...] = y


def _tc_s23(y1, wt_as, b_as, wt_dn, b_dn, ids_as, ids_dn):
    full = lambda shape: pl.BlockSpec(shape, lambda i: (0,) * len(shape))
    return pl.pallas_call(
        _s23_body,
        grid=(TC_GRID,),
        in_specs=[
            pl.BlockSpec((TC_BLK, D), lambda i: (i, 0)),
            full(wt_as.shape), full(b_as.shape),
            full(wt_dn.shape), full(b_dn.shape),
            pl.BlockSpec((1, TC_BLK, 1), lambda i: (i, 0, 0)),
            pl.BlockSpec((1, TC_BLK, 1), lambda i: (i, 0, 0)),
        ],
        out_specs=pl.BlockSpec((TC_BLK, D), lambda i: (i, 0)),
        out_shape=jax.ShapeDtypeStruct((N, D), jnp.float32),
    )(y1, wt_as, b_as, wt_dn, b_dn, ids_as, ids_dn)


@jax.jit
def kernel(z, W_ds, b_ds, W_as, b_as, W_dn, b_dn, dataset_id, assay_id, donor_id):
    ds_id = dataset_id.astype(jnp.int32)
    zs, posback, eot = _sc_sort_scatter(ds_id, z)
    y1s = _tc_grouped_stage1(eot, zs, W_ds.astype(jnp.bfloat16), b_ds)
    y1 = _sc_gather(posback, y1s)
    wt_as = W_as.reshape(-1, D).astype(jnp.bfloat16)
    wt_dn = W_dn.reshape(-1, D).astype(jnp.bfloat16)
    ids_as = assay_id.astype(jnp.int32).reshape(TC_GRID, TC_BLK, 1)
    ids_dn = donor_id.astype(jnp.int32).reshape(TC_GRID, TC_BLK, 1)
    return _tc_s23(y1, wt_as, b_as, wt_dn, b_dn, ids_as, ids_dn)
